# Initial kernel scaffold; baseline (speedup 1.0000x reference)
import jax
import jax.numpy as jnp
from jax.experimental import pallas as pl


def kernel(h, edge_index, edge_attr, W_lin, bias, W1, b1, W2, b2):
    raise NotImplementedError("write your pallas kernel here")


# speedup vs baseline: 10.7025x; 10.7025x over previous
import functools

import jax
import jax.numpy as jnp
from jax import lax
from jax.experimental import pallas as pl
from jax.experimental.pallas import tpu as pltpu
from jax.experimental.pallas import tpu_sc as plsc

_N = 10000
_E = 320000
_HID = 128

_BE = 3200


def _mlp_body(ea_ref, W1_ref, b1_ref, W2_ref, b2_ref, o_ref):
    ea = ea_ref[...]
    W1 = W1_ref[...]
    z = b1_ref[...] + (W1[:, 0:1] * ea[0:1, :] + W1[:, 1:2] * ea[1:2, :]
                       + W1[:, 2:3] * ea[2:3, :] + W1[:, 3:4] * ea[3:4, :])
    z = z * jax.nn.sigmoid(z)
    s = jnp.sum(z * W2_ref[...], axis=0, keepdims=True) + b2_ref[...]
    o_ref[...] = jax.nn.sigmoid(s)


def _edge_mlp(edge_attr, W1, b1, W2, b2):
    eaT = edge_attr.T
    w = pl.pallas_call(
        _mlp_body,
        grid=(_E // _BE,),
        in_specs=[
            pl.BlockSpec((4, _BE), lambda i: (0, i)),
            pl.BlockSpec((64, 4), lambda i: (0, 0)),
            pl.BlockSpec((64, 1), lambda i: (0, 0)),
            pl.BlockSpec((64, 1), lambda i: (0, 0)),
            pl.BlockSpec((1, 1), lambda i: (0, 0)),
        ],
        out_specs=pl.BlockSpec((1, _BE), lambda i: (0, i)),
        out_shape=jax.ShapeDtypeStruct((1, _E), jnp.float32),
    )(eaT, W1, b1.reshape(64, 1), W2.reshape(64, 1), b2.reshape(1, 1))
    return w.reshape(_E)


_BN = 400


def _lin_body(h_ref, w_ref, o_ref):
    o_ref[...] = jnp.dot(h_ref[...], w_ref[...],
                         preferred_element_type=jnp.float32)


def _linear(h, W_lin):
    return pl.pallas_call(
        _lin_body,
        grid=(_N // _BN,),
        in_specs=[
            pl.BlockSpec((_BN, _HID), lambda i: (i, 0)),
            pl.BlockSpec((_HID, _HID), lambda i: (0, 0)),
        ],
        out_specs=pl.BlockSpec((_BN, _HID), lambda i: (i, 0)),
        out_shape=jax.ShapeDtypeStruct((_N, _HID), jnp.float32),
    )(h, W_lin.T)


_NC = 2
_NS = 16
_C = 80
_EPT = _E // (_NC * _NS)
_ZR = 200
_NZCH = _N // _ZR


def _sc_deg_body(col_hbm, w_hbm, d0_hbm, d1_hbm,
                 deg_spm, zb, colb, wb):
    c = lax.axis_index("c")
    s = lax.axis_index("s")

    def _z16(i, _):
        zb[pl.ds(i * 16, 16)] = jnp.zeros((16,), jnp.float32)
        return 0
    lax.fori_loop(0, _N // 16, _z16, 0)

    @pl.when(s == 0)
    def _():
        pltpu.sync_copy(zb, deg_spm)
    plsc.subcore_barrier()

    ebase = c * (_E // _NC) + s * _EPT

    def _deg_step(i, _):
        off = ebase + i * _C
        pltpu.sync_copy(col_hbm.at[pl.ds(off, _C)], colb)
        pltpu.sync_copy(w_hbm.at[pl.ds(off, _C)], wb)
        pltpu.sync_copy(wb, deg_spm.at[colb], add=True)
        return 0
    lax.fori_loop(0, _EPT // _C, _deg_step, 0)
    plsc.subcore_barrier()

    @pl.when(jnp.logical_and(c == 0, s == 0))
    def _():
        pltpu.sync_copy(deg_spm, d0_hbm)

    @pl.when(jnp.logical_and(c == 1, s == 0))
    def _():
        pltpu.sync_copy(deg_spm, d1_hbm)


def _sc_deg(col, w):
    mesh = plsc.VectorSubcoreMesh(core_axis_name="c", subcore_axis_name="s")
    f = pl.kernel(
        _sc_deg_body,
        out_type=(
            jax.ShapeDtypeStruct((_N,), jnp.float32),
            jax.ShapeDtypeStruct((_N,), jnp.float32),
        ),
        mesh=mesh,
        compiler_params=pltpu.CompilerParams(needs_layout_passes=False),
        scratch_types=[
            pltpu.VMEM_SHARED((_N,), jnp.float32),
            pltpu.VMEM((_N,), jnp.float32),
            pltpu.VMEM((_C,), jnp.int32),
            pltpu.VMEM((_C,), jnp.float32),
        ],
    )
    return f(col, w)


def _rsqrt_body(d0_ref, d1_ref, o_ref):
    o_ref[...] = lax.rsqrt(d0_ref[...] + d1_ref[...] + 1.0)


def _deg_rsqrt(d0, d1):
    dis = pl.pallas_call(
        _rsqrt_body,
        out_shape=jax.ShapeDtypeStruct((100, 100), jnp.float32),
    )(d0.reshape(100, 100), d1.reshape(100, 100))
    return dis.reshape(_N)


def _sc_main_body(row_hbm, col_hbm, w_hbm, dis_hbm, xl_hbm, p0_hbm, p1_hbm,
                  out_spm, dis_v, zrows, rows, rowb, colb, wb, normb, sem):
    c = lax.axis_index("c")
    s = lax.axis_index("s")

    def _zr(i, _):
        for q in range(_HID // 16):
            zrows[i, pl.ds(q * 16, 16)] = jnp.zeros((16,), jnp.float32)
        return 0
    lax.fori_loop(0, _ZR, _zr, 0)

    pltpu.sync_copy(dis_hbm, dis_v)

    for k in range((_NZCH + _NS - 1) // _NS):
        ch = s + k * _NS

        @pl.when(ch < _NZCH)
        def _():
            pltpu.sync_copy(zrows, out_spm.at[pl.ds(ch * _ZR, _ZR)])
    plsc.subcore_barrier()

    ebase = c * (_E // _NC) + s * _EPT

    def _main_step(i, _):
        off = ebase + i * _C
        pltpu.sync_copy(row_hbm.at[pl.ds(off, _C)], rowb)
        pltpu.sync_copy(col_hbm.at[pl.ds(off, _C)], colb)
        pltpu.sync_copy(w_hbm.at[pl.ds(off, _C)], wb)
        cp = pltpu.async_copy(xl_hbm.at[rowb], rows, sem)
        for k in range(_C // 16):
            r16 = rowb[pl.ds(k * 16, 16)]
            c16 = colb[pl.ds(k * 16, 16)]
            dr = plsc.load_gather(dis_v, [r16])
            dc = plsc.load_gather(dis_v, [c16])
            normb[pl.ds(k * 16, 16)] = dr * wb[pl.ds(k * 16, 16)] * dc
        cp.wait()

        def _scale(j, _2):
            sv = plsc.load_gather(normb, [jnp.broadcast_to(j, (16,))])
            for q in range(_HID // 16):
                rows[j, pl.ds(q * 16, 16)] = rows[j, pl.ds(q * 16, 16)] * sv
            return 0
        lax.fori_loop(0, _C, _scale, 0)
        pltpu.sync_copy(rows, out_spm.at[colb], add=True)
        return 0
    lax.fori_loop(0, _EPT // _C, _main_step, 0)
    plsc.subcore_barrier()

    for k in range((_NZCH + _NS - 1) // _NS):
        ch = s + k * _NS

        @pl.when(jnp.logical_and(c == 0, ch < _NZCH))
        def _():
            o = ch * _ZR
            pltpu.sync_copy(out_spm.at[pl.ds(o, _ZR)], p0_hbm.at[pl.ds(o, _ZR)])

        @pl.when(jnp.logical_and(c == 1, ch < _NZCH))
        def _():
            o = ch * _ZR
            pltpu.sync_copy(out_spm.at[pl.ds(o, _ZR)], p1_hbm.at[pl.ds(o, _ZR)])


def _sc_main(row, col, w, dis, xl):
    mesh = plsc.VectorSubcoreMesh(core_axis_name="c", subcore_axis_name="s")
    f = pl.kernel(
        _sc_main_body,
        out_type=(
            jax.ShapeDtypeStruct((_N, _HID), jnp.float32),
            jax.ShapeDtypeStruct((_N, _HID), jnp.float32),
        ),
        mesh=mesh,
        compiler_params=pltpu.CompilerParams(needs_layout_passes=False),
        scratch_types=[
            pltpu.VMEM_SHARED((_N, _HID), jnp.float32),
            pltpu.VMEM((_N,), jnp.float32),
            pltpu.VMEM((_ZR, _HID), jnp.float32),
            pltpu.VMEM((_C, _HID), jnp.float32),
            pltpu.VMEM((_C,), jnp.int32),
            pltpu.VMEM((_C,), jnp.int32),
            pltpu.VMEM((_C,), jnp.float32),
            pltpu.VMEM((_C,), jnp.float32),
            pltpu.SemaphoreType.DMA,
        ],
    )
    return f(row, col, w, dis, xl)


def _final_body(p0_ref, p1_ref, xl_ref, dis_ref, b_ref, o_ref):
    d = dis_ref[...]
    o_ref[...] = p0_ref[...] + p1_ref[...] + d * d * xl_ref[...] + b_ref[...]


def _final(p0, p1, xl, dis, bias):
    return pl.pallas_call(
        _final_body,
        grid=(_N // _BN,),
        in_specs=[
            pl.BlockSpec((_BN, _HID), lambda i: (i, 0)),
            pl.BlockSpec((_BN, _HID), lambda i: (i, 0)),
            pl.BlockSpec((_BN, _HID), lambda i: (i, 0)),
            pl.BlockSpec((_BN, 1), lambda i: (i, 0)),
            pl.BlockSpec((1, _HID), lambda i: (0, 0)),
        ],
        out_specs=pl.BlockSpec((_BN, _HID), lambda i: (i, 0)),
        out_shape=jax.ShapeDtypeStruct((_N, _HID), jnp.float32),
    )(p0, p1, xl, dis.reshape(_N, 1), bias.reshape(1, _HID))


def kernel(h, edge_index, edge_attr, W_lin, bias, W1, b1, W2, b2):
    row = edge_index[0]
    col = edge_index[1]
    w = _edge_mlp(edge_attr, W1, b1, W2, b2)
    xl = _linear(h, W_lin)
    d0, d1 = _sc_deg(col, w)
    dis = _deg_rsqrt(d0, d1)
    p0, p1 = _sc_main(row, col, w, dis, xl)
    return _final(p0, p1, xl, dis, bias)


# speedup vs baseline: 12.8786x; 1.2033x over previous
import functools

import jax
import jax.numpy as jnp
from jax import lax
from jax.experimental import pallas as pl
from jax.experimental.pallas import tpu as pltpu
from jax.experimental.pallas import tpu_sc as plsc

_N = 10000
_E = 320000
_HID = 128

_BE = 3200


def _mlp_body(ea_ref, W1_ref, b1_ref, W2_ref, b2_ref, o_ref):
    ea = ea_ref[...]
    W1 = W1_ref[...]
    z = b1_ref[...] + (W1[:, 0:1] * ea[0:1, :] + W1[:, 1:2] * ea[1:2, :]
                       + W1[:, 2:3] * ea[2:3, :] + W1[:, 3:4] * ea[3:4, :])
    z = z * jax.nn.sigmoid(z)
    s = jnp.sum(z * W2_ref[...], axis=0, keepdims=True) + b2_ref[...]
    o_ref[...] = jax.nn.sigmoid(s)


def _edge_mlp(edge_attr, W1, b1, W2, b2):
    eaT = edge_attr.T
    w = pl.pallas_call(
        _mlp_body,
        grid=(_E // _BE,),
        in_specs=[
            pl.BlockSpec((4, _BE), lambda i: (0, i)),
            pl.BlockSpec((64, 4), lambda i: (0, 0)),
            pl.BlockSpec((64, 1), lambda i: (0, 0)),
            pl.BlockSpec((64, 1), lambda i: (0, 0)),
            pl.BlockSpec((1, 1), lambda i: (0, 0)),
        ],
        out_specs=pl.BlockSpec((1, _BE), lambda i: (0, i)),
        out_shape=jax.ShapeDtypeStruct((1, _E), jnp.float32),
    )(eaT, W1, b1.reshape(64, 1), W2.reshape(64, 1), b2.reshape(1, 1))
    return w.reshape(_E)


_BN = 400


def _lin_body(h_ref, w_ref, o_ref):
    o_ref[...] = jnp.dot(h_ref[...], w_ref[...],
                         preferred_element_type=jnp.float32)


def _linear(h, W_lin):
    return pl.pallas_call(
        _lin_body,
        grid=(_N // _BN,),
        in_specs=[
            pl.BlockSpec((_BN, _HID), lambda i: (i, 0)),
            pl.BlockSpec((_HID, _HID), lambda i: (0, 0)),
        ],
        out_specs=pl.BlockSpec((_BN, _HID), lambda i: (i, 0)),
        out_shape=jax.ShapeDtypeStruct((_N, _HID), jnp.float32),
    )(h, W_lin.T)


_NC = 2
_NS = 16
_C = 80
_EPT = _E // (_NC * _NS)
_ZR = 80
_NZCH = _N // _ZR


def _sc_deg_body(col_hbm, w_hbm, d0_hbm, d1_hbm,
                 deg_spm, zb, colb0, wb0, colb1, wb1, dsem0, dsem1):
    c = lax.axis_index("c")
    s = lax.axis_index("s")

    def _z16(i, _):
        zb[pl.ds(i * 16, 16)] = jnp.zeros((16,), jnp.float32)
        return 0
    lax.fori_loop(0, _N // 16, _z16, 0)

    @pl.when(s == 0)
    def _():
        pltpu.sync_copy(zb, deg_spm)
    plsc.subcore_barrier()

    ebase = c * (_E // _NC) + s * _EPT
    bufs = ((colb0, wb0, dsem0), (colb1, wb1, dsem1))
    nch = _EPT // _C

    def _fetch(ch, bi):
        colb, wb, _sm = bufs[bi]
        off = ebase + ch * _C
        pltpu.sync_copy(col_hbm.at[pl.ds(off, _C)], colb)
        pltpu.sync_copy(w_hbm.at[pl.ds(off, _C)], wb)

    def _scat(bi):
        colb, wb, sm = bufs[bi]
        return pltpu.async_copy(wb, deg_spm.at[colb], add=True, sem=sm)

    def _scat_wait(bi):
        colb, wb, sm = bufs[bi]
        pltpu.make_async_copy(wb, deg_spm.at[colb], sm).wait()

    _fetch(0, 0)

    def _deg_pair(i, _):
        for b in (0, 1):
            ch = 2 * i + b
            if b == 0:
                @pl.when(i > 0)
                def _():
                    _scat_wait(1)
            else:
                _scat_wait(0)
            _fetch(ch + 1, 1 - b)
            _scat(b)
        return 0
    lax.fori_loop(0, nch // 2, _deg_pair, 0)
    _scat_wait(1)
    _scat(0)
    _scat_wait(0)
    plsc.subcore_barrier()

    @pl.when(jnp.logical_and(c == 0, s == 0))
    def _():
        pltpu.sync_copy(deg_spm, d0_hbm)

    @pl.when(jnp.logical_and(c == 1, s == 0))
    def _():
        pltpu.sync_copy(deg_spm, d1_hbm)


def _sc_deg(col, w):
    mesh = plsc.VectorSubcoreMesh(core_axis_name="c", subcore_axis_name="s")
    f = pl.kernel(
        _sc_deg_body,
        out_type=(
            jax.ShapeDtypeStruct((_N,), jnp.float32),
            jax.ShapeDtypeStruct((_N,), jnp.float32),
        ),
        mesh=mesh,
        compiler_params=pltpu.CompilerParams(needs_layout_passes=False),
        scratch_types=[
            pltpu.VMEM_SHARED((_N,), jnp.float32),
            pltpu.VMEM((_N,), jnp.float32),
            pltpu.VMEM((_C,), jnp.int32),
            pltpu.VMEM((_C,), jnp.float32),
            pltpu.VMEM((_C,), jnp.int32),
            pltpu.VMEM((_C,), jnp.float32),
            pltpu.SemaphoreType.DMA,
            pltpu.SemaphoreType.DMA,
        ],
    )
    return f(col, w)


def _rsqrt_body(d0_ref, d1_ref, o_ref):
    o_ref[...] = lax.rsqrt(d0_ref[...] + d1_ref[...] + 1.0)


def _deg_rsqrt(d0, d1):
    dis = pl.pallas_call(
        _rsqrt_body,
        out_shape=jax.ShapeDtypeStruct((100, 100), jnp.float32),
    )(d0.reshape(100, 100), d1.reshape(100, 100))
    return dis.reshape(_N)


def _sc_main_body(row_hbm, col_hbm, w_hbm, dis_hbm, xl_hbm, p0_hbm, p1_hbm,
                  out_spm, dis_v,
                  rows0, rowb0, colb0, wb0, normb0, gsem0, ssem0,
                  rows1, rowb1, colb1, wb1, normb1, gsem1, ssem1):
    c = lax.axis_index("c")
    s = lax.axis_index("s")

    def _zr(i, _):
        for q in range(_HID // 16):
            rows0[i, pl.ds(q * 16, 16)] = jnp.zeros((16,), jnp.float32)
        return 0
    lax.fori_loop(0, _ZR, _zr, 0)

    pltpu.sync_copy(dis_hbm, dis_v)

    for k in range((_NZCH + _NS - 1) // _NS):
        ch = s + k * _NS

        @pl.when(ch < _NZCH)
        def _():
            pltpu.sync_copy(rows0, out_spm.at[pl.ds(ch * _ZR, _ZR)])
    plsc.subcore_barrier()

    ebase = c * (_E // _NC) + s * _EPT
    bufs = ((rows0, rowb0, colb0, wb0, normb0, gsem0, ssem0),
            (rows1, rowb1, colb1, wb1, normb1, gsem1, ssem1))
    nch = _EPT // _C

    def _issue(ch, bi):
        rows, rowb, colb, wb, _n, gsem, _s = bufs[bi]
        off = ebase + ch * _C
        pltpu.sync_copy(row_hbm.at[pl.ds(off, _C)], rowb)
        pltpu.sync_copy(col_hbm.at[pl.ds(off, _C)], colb)
        pltpu.sync_copy(w_hbm.at[pl.ds(off, _C)], wb)
        pltpu.async_copy(xl_hbm.at[rowb], rows, gsem)

    def _gwait(bi):
        rows, rowb, _c, _w, _n, gsem, _s = bufs[bi]
        pltpu.make_async_copy(xl_hbm.at[rowb], rows, gsem).wait()

    def _swait(bi):
        rows, _r, colb, _w, _n, _g, ssem = bufs[bi]
        pltpu.make_async_copy(rows, out_spm.at[colb], ssem).wait()

    def _compute(bi):
        rows, rowb, colb, wb, normb, _g, ssem = bufs[bi]
        for k in range(_C // 16):
            r16 = rowb[pl.ds(k * 16, 16)]
            c16 = colb[pl.ds(k * 16, 16)]
            dr = plsc.load_gather(dis_v, [r16])
            dc = plsc.load_gather(dis_v, [c16])
            normb[pl.ds(k * 16, 16)] = dr * wb[pl.ds(k * 16, 16)] * dc

        def _scale(j, _2):
            sv = plsc.load_gather(normb, [jnp.broadcast_to(j, (16,))])
            for q in range(_HID // 16):
                rows[j, pl.ds(q * 16, 16)] = rows[j, pl.ds(q * 16, 16)] * sv
            return 0
        lax.fori_loop(0, _C, _scale, 0)
        pltpu.async_copy(rows, out_spm.at[colb], add=True, sem=ssem)

    _issue(0, 0)

    def _main_pair(i, _):
        for b in (0, 1):
            ch = 2 * i + b
            _gwait(b)
            if b == 0:
                @pl.when(i > 0)
                def _():
                    _swait(1)
            else:
                _swait(0)
            _issue(ch + 1, 1 - b)
            _compute(b)
        return 0
    lax.fori_loop(0, nch // 2, _main_pair, 0)
    _gwait(0)
    _swait(1)
    _compute(0)
    _swait(0)
    plsc.subcore_barrier()

    for k in range((_NZCH + _NS - 1) // _NS):
        ch = s + k * _NS

        @pl.when(jnp.logical_and(c == 0, ch < _NZCH))
        def _():
            o = ch * _ZR
            pltpu.sync_copy(out_spm.at[pl.ds(o, _ZR)], p0_hbm.at[pl.ds(o, _ZR)])

        @pl.when(jnp.logical_and(c == 1, ch < _NZCH))
        def _():
            o = ch * _ZR
            pltpu.sync_copy(out_spm.at[pl.ds(o, _ZR)], p1_hbm.at[pl.ds(o, _ZR)])


def _sc_main(row, col, w, dis, xl):
    mesh = plsc.VectorSubcoreMesh(core_axis_name="c", subcore_axis_name="s")
    f = pl.kernel(
        _sc_main_body,
        out_type=(
            jax.ShapeDtypeStruct((_N, _HID), jnp.float32),
            jax.ShapeDtypeStruct((_N, _HID), jnp.float32),
        ),
        mesh=mesh,
        compiler_params=pltpu.CompilerParams(needs_layout_passes=False),
        scratch_types=[
            pltpu.VMEM_SHARED((_N, _HID), jnp.float32),
            pltpu.VMEM((_N,), jnp.float32),
            pltpu.VMEM((_C, _HID), jnp.float32),
            pltpu.VMEM((_C,), jnp.int32),
            pltpu.VMEM((_C,), jnp.int32),
            pltpu.VMEM((_C,), jnp.float32),
            pltpu.VMEM((_C,), jnp.float32),
            pltpu.SemaphoreType.DMA,
            pltpu.SemaphoreType.DMA,
            pltpu.VMEM((_C, _HID), jnp.float32),
            pltpu.VMEM((_C,), jnp.int32),
            pltpu.VMEM((_C,), jnp.int32),
            pltpu.VMEM((_C,), jnp.float32),
            pltpu.VMEM((_C,), jnp.float32),
            pltpu.SemaphoreType.DMA,
            pltpu.SemaphoreType.DMA,
        ],
    )
    return f(row, col, w, dis, xl)


def _final_body(p0_ref, p1_ref, xl_ref, dis_ref, b_ref, o_ref):
    d = dis_ref[...]
    o_ref[...] = p0_ref[...] + p1_ref[...] + d * d * xl_ref[...] + b_ref[...]


def _final(p0, p1, xl, dis, bias):
    return pl.pallas_call(
        _final_body,
        grid=(_N // _BN,),
        in_specs=[
            pl.BlockSpec((_BN, _HID), lambda i: (i, 0)),
            pl.BlockSpec((_BN, _HID), lambda i: (i, 0)),
            pl.BlockSpec((_BN, _HID), lambda i: (i, 0)),
            pl.BlockSpec((_BN, 1), lambda i: (i, 0)),
            pl.BlockSpec((1, _HID), lambda i: (0, 0)),
        ],
        out_specs=pl.BlockSpec((_BN, _HID), lambda i: (i, 0)),
        out_shape=jax.ShapeDtypeStruct((_N, _HID), jnp.float32),
    )(p0, p1, xl, dis.reshape(_N, 1), bias.reshape(1, _HID))


def kernel(h, edge_index, edge_attr, W_lin, bias, W1, b1, W2, b2):
    row = edge_index[0]
    col = edge_index[1]
    w = _edge_mlp(edge_attr, W1, b1, W2, b2)
    xl = _linear(h, W_lin)
    d0, d1 = _sc_deg(col, w)
    dis = _deg_rsqrt(d0, d1)
    p0, p1 = _sc_main(row, col, w, dis, xl)
    return _final(p0, p1, xl, dis, bias)


# speedup vs baseline: 13.0600x; 1.0141x over previous
import functools

import jax
import jax.numpy as jnp
from jax import lax
from jax.experimental import pallas as pl
from jax.experimental.pallas import tpu as pltpu
from jax.experimental.pallas import tpu_sc as plsc

_N = 10000
_E = 320000
_HID = 128

_BE = 3200


def _mlp_body(ea_ref, W1_ref, b1_ref, W2_ref, b2_ref, o_ref):
    ea = ea_ref[...]
    W1 = W1_ref[...]
    z = b1_ref[...] + (W1[:, 0:1] * ea[0:1, :] + W1[:, 1:2] * ea[1:2, :]
                       + W1[:, 2:3] * ea[2:3, :] + W1[:, 3:4] * ea[3:4, :])
    z = z * jax.nn.sigmoid(z)
    s = jnp.sum(z * W2_ref[...], axis=0, keepdims=True) + b2_ref[...]
    o_ref[...] = jax.nn.sigmoid(s)


def _edge_mlp(edge_attr, W1, b1, W2, b2):
    eaT = edge_attr.T
    w = pl.pallas_call(
        _mlp_body,
        grid=(_E // _BE,),
        in_specs=[
            pl.BlockSpec((4, _BE), lambda i: (0, i)),
            pl.BlockSpec((64, 4), lambda i: (0, 0)),
            pl.BlockSpec((64, 1), lambda i: (0, 0)),
            pl.BlockSpec((64, 1), lambda i: (0, 0)),
            pl.BlockSpec((1, 1), lambda i: (0, 0)),
        ],
        out_specs=pl.BlockSpec((1, _BE), lambda i: (0, i)),
        out_shape=jax.ShapeDtypeStruct((1, _E), jnp.float32),
    )(eaT, W1, b1.reshape(64, 1), W2.reshape(64, 1), b2.reshape(1, 1))
    return w.reshape(_E)


_BN = 400


def _lin_body(h_ref, w_ref, o_ref):
    o_ref[...] = jnp.dot(h_ref[...], w_ref[...],
                         preferred_element_type=jnp.float32)


def _linear(h, W_lin):
    return pl.pallas_call(
        _lin_body,
        grid=(_N // _BN,),
        in_specs=[
            pl.BlockSpec((_BN, _HID), lambda i: (i, 0)),
            pl.BlockSpec((_HID, _HID), lambda i: (0, 0)),
        ],
        out_specs=pl.BlockSpec((_BN, _HID), lambda i: (i, 0)),
        out_shape=jax.ShapeDtypeStruct((_N, _HID), jnp.float32),
    )(h, W_lin.T)


_NC = 2
_NS = 16
_C = 80
_EPT = _E // (_NC * _NS)
_ZR = 80
_NZCH = _N // _ZR


def _sc_deg_body(col_hbm, w_hbm, d0_hbm, d1_hbm,
                 deg_spm, zb, colb0, wb0, colb1, wb1, dsem0, dsem1):
    c = lax.axis_index("c")
    s = lax.axis_index("s")

    def _z16(i, _):
        zb[pl.ds(i * 16, 16)] = jnp.zeros((16,), jnp.float32)
        return 0
    lax.fori_loop(0, _N // 16, _z16, 0)

    @pl.when(s == 0)
    def _():
        pltpu.sync_copy(zb, deg_spm)
    plsc.subcore_barrier()

    ebase = c * (_E // _NC) + s * _EPT
    bufs = ((colb0, wb0, dsem0), (colb1, wb1, dsem1))
    nch = _EPT // _C

    def _fetch(ch, bi):
        colb, wb, _sm = bufs[bi]
        off = ebase + ch * _C
        pltpu.sync_copy(col_hbm.at[pl.ds(off, _C)], colb)
        pltpu.sync_copy(w_hbm.at[pl.ds(off, _C)], wb)

    def _scat(bi):
        colb, wb, sm = bufs[bi]
        return pltpu.async_copy(wb, deg_spm.at[colb], add=True, sem=sm)

    def _scat_wait(bi):
        colb, wb, sm = bufs[bi]
        pltpu.make_async_copy(wb, deg_spm.at[colb], sm).wait()

    _fetch(0, 0)

    def _deg_pair(i, _):
        for b in (0, 1):
            ch = 2 * i + b
            if b == 0:
                @pl.when(i > 0)
                def _():
                    _scat_wait(1)
            else:
                _scat_wait(0)
            _fetch(ch + 1, 1 - b)
            _scat(b)
        return 0
    lax.fori_loop(0, nch // 2, _deg_pair, 0)
    _scat_wait(1)
    _scat(0)
    _scat_wait(0)
    plsc.subcore_barrier()

    @pl.when(jnp.logical_and(c == 0, s == 0))
    def _():
        pltpu.sync_copy(deg_spm, d0_hbm)

    @pl.when(jnp.logical_and(c == 1, s == 0))
    def _():
        pltpu.sync_copy(deg_spm, d1_hbm)


def _sc_deg(col, w):
    mesh = plsc.VectorSubcoreMesh(core_axis_name="c", subcore_axis_name="s")
    f = pl.kernel(
        _sc_deg_body,
        out_type=(
            jax.ShapeDtypeStruct((_N,), jnp.float32),
            jax.ShapeDtypeStruct((_N,), jnp.float32),
        ),
        mesh=mesh,
        compiler_params=pltpu.CompilerParams(needs_layout_passes=False),
        scratch_types=[
            pltpu.VMEM_SHARED((_N,), jnp.float32),
            pltpu.VMEM((_N,), jnp.float32),
            pltpu.VMEM((_C,), jnp.int32),
            pltpu.VMEM((_C,), jnp.float32),
            pltpu.VMEM((_C,), jnp.int32),
            pltpu.VMEM((_C,), jnp.float32),
            pltpu.SemaphoreType.DMA,
            pltpu.SemaphoreType.DMA,
        ],
    )
    return f(col, w)


def _rsqrt_body(d0_ref, d1_ref, o_ref):
    o_ref[...] = lax.rsqrt(d0_ref[...] + d1_ref[...] + 1.0)


def _deg_rsqrt(d0, d1):
    dis = pl.pallas_call(
        _rsqrt_body,
        out_shape=jax.ShapeDtypeStruct((100, 100), jnp.float32),
    )(d0.reshape(100, 100), d1.reshape(100, 100))
    return dis.reshape(_N)


def _sc_main_body(row_hbm, col_hbm, w_hbm, dis_hbm, xl_hbm, p0_hbm, p1_hbm,
                  out_spm, dis_v,
                  rows0, rowb0, colb0, wb0, normb0, gsem0, ssem0,
                  rows1, rowb1, colb1, wb1, normb1, gsem1, ssem1):
    c = lax.axis_index("c")
    s = lax.axis_index("s")

    def _zr(i, _):
        for q in range(_HID // 16):
            rows0[i, pl.ds(q * 16, 16)] = jnp.zeros((16,), jnp.float32)
        return 0
    lax.fori_loop(0, _ZR, _zr, 0)

    pltpu.sync_copy(dis_hbm, dis_v)

    for k in range((_NZCH + _NS - 1) // _NS):
        ch = s + k * _NS

        @pl.when(ch < _NZCH)
        def _():
            pltpu.sync_copy(rows0, out_spm.at[pl.ds(ch * _ZR, _ZR)])
    plsc.subcore_barrier()

    ebase = c * (_E // _NC) + s * _EPT
    bufs = ((rows0, rowb0, colb0, wb0, normb0, gsem0, ssem0),
            (rows1, rowb1, colb1, wb1, normb1, gsem1, ssem1))
    nch = _EPT // _C

    def _issue(ch, bi):
        rows, rowb, colb, wb, _n, gsem, _s = bufs[bi]
        off = ebase + ch * _C
        pltpu.sync_copy(row_hbm.at[pl.ds(off, _C)], rowb)
        pltpu.sync_copy(col_hbm.at[pl.ds(off, _C)], colb)
        pltpu.sync_copy(w_hbm.at[pl.ds(off, _C)], wb)
        pltpu.async_copy(xl_hbm.at[rowb], rows, gsem)

    def _gwait(bi):
        rows, rowb, _c, _w, _n, gsem, _s = bufs[bi]
        pltpu.make_async_copy(xl_hbm.at[rowb], rows, gsem).wait()

    def _swait(bi):
        rows, _r, colb, _w, _n, _g, ssem = bufs[bi]
        pltpu.make_async_copy(rows, out_spm.at[colb], ssem).wait()

    def _compute(bi):
        rows, rowb, colb, wb, normb, _g, ssem = bufs[bi]
        for k in range(_C // 16):
            r16 = rowb[pl.ds(k * 16, 16)]
            c16 = colb[pl.ds(k * 16, 16)]
            dr = plsc.load_gather(dis_v, [r16])
            dc = plsc.load_gather(dis_v, [c16])
            normb[pl.ds(k * 16, 16)] = dr * wb[pl.ds(k * 16, 16)] * dc

        def _scale(j, _2):
            j0 = 4 * j
            for u in range(4):
                sv = plsc.load_gather(normb, [jnp.broadcast_to(j0 + u, (16,))])
                for q in range(_HID // 16):
                    rows[j0 + u, pl.ds(q * 16, 16)] = (
                        rows[j0 + u, pl.ds(q * 16, 16)] * sv)
            return 0
        lax.fori_loop(0, _C // 4, _scale, 0)
        pltpu.async_copy(rows, out_spm.at[colb], add=True, sem=ssem)

    _issue(0, 0)

    def _main_pair(i, _):
        for b in (0, 1):
            ch = 2 * i + b
            _gwait(b)
            if b == 0:
                @pl.when(i > 0)
                def _():
                    _swait(1)
            else:
                _swait(0)
            _issue(ch + 1, 1 - b)
            _compute(b)
        return 0
    lax.fori_loop(0, nch // 2, _main_pair, 0)
    _gwait(0)
    _swait(1)
    _compute(0)
    _swait(0)
    plsc.subcore_barrier()

    for k in range((_NZCH + _NS - 1) // _NS):
        ch = s + k * _NS

        @pl.when(jnp.logical_and(c == 0, ch < _NZCH))
        def _():
            o = ch * _ZR
            pltpu.sync_copy(out_spm.at[pl.ds(o, _ZR)], p0_hbm.at[pl.ds(o, _ZR)])

        @pl.when(jnp.logical_and(c == 1, ch < _NZCH))
        def _():
            o = ch * _ZR
            pltpu.sync_copy(out_spm.at[pl.ds(o, _ZR)], p1_hbm.at[pl.ds(o, _ZR)])


def _sc_main(row, col, w, dis, xl):
    mesh = plsc.VectorSubcoreMesh(core_axis_name="c", subcore_axis_name="s")
    f = pl.kernel(
        _sc_main_body,
        out_type=(
            jax.ShapeDtypeStruct((_N, _HID), jnp.float32),
            jax.ShapeDtypeStruct((_N, _HID), jnp.float32),
        ),
        mesh=mesh,
        compiler_params=pltpu.CompilerParams(needs_layout_passes=False),
        scratch_types=[
            pltpu.VMEM_SHARED((_N, _HID), jnp.float32),
            pltpu.VMEM((_N,), jnp.float32),
            pltpu.VMEM((_C, _HID), jnp.float32),
            pltpu.VMEM((_C,), jnp.int32),
            pltpu.VMEM((_C,), jnp.int32),
            pltpu.VMEM((_C,), jnp.float32),
            pltpu.VMEM((_C,), jnp.float32),
            pltpu.SemaphoreType.DMA,
            pltpu.SemaphoreType.DMA,
            pltpu.VMEM((_C, _HID), jnp.float32),
            pltpu.VMEM((_C,), jnp.int32),
            pltpu.VMEM((_C,), jnp.int32),
            pltpu.VMEM((_C,), jnp.float32),
            pltpu.VMEM((_C,), jnp.float32),
            pltpu.SemaphoreType.DMA,
            pltpu.SemaphoreType.DMA,
        ],
    )
    return f(row, col, w, dis, xl)


def _final_body(p0_ref, p1_ref, xl_ref, dis_ref, b_ref, o_ref):
    d = dis_ref[...]
    o_ref[...] = p0_ref[...] + p1_ref[...] + d * d * xl_ref[...] + b_ref[...]


def _final(p0, p1, xl, dis, bias):
    return pl.pallas_call(
        _final_body,
        grid=(_N // _BN,),
        in_specs=[
            pl.BlockSpec((_BN, _HID), lambda i: (i, 0)),
            pl.BlockSpec((_BN, _HID), lambda i: (i, 0)),
            pl.BlockSpec((_BN, _HID), lambda i: (i, 0)),
            pl.BlockSpec((_BN, 1), lambda i: (i, 0)),
            pl.BlockSpec((1, _HID), lambda i: (0, 0)),
        ],
        out_specs=pl.BlockSpec((_BN, _HID), lambda i: (i, 0)),
        out_shape=jax.ShapeDtypeStruct((_N, _HID), jnp.float32),
    )(p0, p1, xl, dis.reshape(_N, 1), bias.reshape(1, _HID))


def kernel(h, edge_index, edge_attr, W_lin, bias, W1, b1, W2, b2):
    row = edge_index[0]
    col = edge_index[1]
    w = _edge_mlp(edge_attr, W1, b1, W2, b2)
    xl = _linear(h, W_lin)
    d0, d1 = _sc_deg(col, w)
    dis = _deg_rsqrt(d0, d1)
    p0, p1 = _sc_main(row, col, w, dis, xl)
    return _final(p0, p1, xl, dis, bias)
